# baseline (speedup 1.0000x reference)
import functools

import jax
import jax.numpy as jnp
from jax import lax
from jax.experimental import pallas as pl
from jax.experimental.pallas import tpu as pltpu
from jax.experimental.pallas import tpu_sc as plsc

_BETA = 0.25
_T = 4096
_KT = 1024


def _combine(lhs, rhs, rounded):
    lv, li = lhs
    rv, ri = rhs
    if rounded:
        u = lax.bitcast_convert_type(lv, jnp.uint32)
        u = (u + jnp.uint32(0x7FFF) + ((u >> 16) & jnp.uint32(1))) \
            & jnp.uint32(0xFFFF0000)
        cmp_l = lax.bitcast_convert_type(u, jnp.float32)
    else:
        cmp_l = lv
    keep = cmp_l <= rv
    return jnp.where(keep, lv, rv), jnp.where(keep, li, ri)


def _argmin_body(n_e, x_ref, sqx_ref, e_ref, sqe_ref, idx_ref, dsum_ref):
    x = x_ref[...]
    xb = x.astype(jnp.bfloat16)
    sqx = sqx_ref[...]
    champs = []
    for c in range(n_e // _KT):
        e_c = e_ref[c * _KT:(c + 1) * _KT, :].astype(jnp.bfloat16)
        sqe_c = sqe_ref[:, c * _KT:(c + 1) * _KT]
        s2 = lax.dot_general(xb, e_c, (((1,), (1,)), ((), ())),
                             preferred_element_type=jnp.float32)
        d = (sqx + sqe_c) + s2
        cm = jnp.min(d, axis=1, keepdims=True)
        io = lax.broadcasted_iota(jnp.int32, (_T, _KT), 1)
        ci = jnp.min(jnp.where(d == cm, io, jnp.int32(2**31 - 1)),
                     axis=1, keepdims=True) + c * _KT
        champs.append((cm, ci))
    a = _combine(champs[0], champs[1], False)
    c = _combine(champs[2], champs[3], False)
    dd = _combine(champs[4], champs[5], False)
    b = _combine(champs[6], champs[7], False)
    a2 = _combine(a, c, False)
    b2 = _combine(dd, b, False)
    m, bi = _combine(a2, b2, True)
    idx_ref[...] = bi

    @pl.when(pl.program_id(0) == 0)
    def _():
        dsum_ref[...] = jnp.zeros_like(dsum_ref)

    dsum_ref[...] += jnp.sum(m, axis=0, keepdims=True)


def _distance_argmin(latent, sqx, emb, sqe):
    n, e_dim = latent.shape
    n_e = emb.shape[0]
    grid = n // _T
    idx, dsum = pl.pallas_call(
        functools.partial(_argmin_body, n_e),
        grid=(grid,),
        in_specs=[
            pl.BlockSpec((_T, e_dim), lambda i: (i, 0)),
            pl.BlockSpec((_T, 1), lambda i: (i, 0)),
            pl.BlockSpec((n_e, e_dim), lambda i: (0, 0)),
            pl.BlockSpec((1, n_e), lambda i: (0, 0)),
        ],
        out_specs=[
            pl.BlockSpec((_T, 1), lambda i: (i, 0)),
            pl.BlockSpec((1, 1), lambda i: (0, 0)),
        ],
        out_shape=[
            jax.ShapeDtypeStruct((n, 1), jnp.int32),
            jax.ShapeDtypeStruct((1, 1), jnp.float32),
        ],
    )(latent, sqx, emb, sqe)
    return idx.reshape(n), dsum[0, 0]


def _sc_gather(table, idx):
    b = idx.shape[0]
    d_orig = table.shape[1]
    table = jnp.pad(table, ((0, 0), (0, 128 - d_orig)))
    d = 128
    info = plsc.get_sparse_core_info()
    nw = info.num_cores * info.num_subcores
    b_per_w = b // nw
    kch = b_per_w // 128
    idx3 = idx.reshape(nw, kch, 128)
    mesh = plsc.VectorSubcoreMesh(core_axis_name="c", subcore_axis_name="s")

    @functools.partial(
        pl.kernel,
        mesh=mesh,
        out_type=jax.ShapeDtypeStruct((b, d), jnp.float32),
        scratch_types=[
            pltpu.VMEM((kch, 128), jnp.int32),
            pltpu.VMEM((b_per_w, d), jnp.float32),
            pltpu.SemaphoreType.DMA,
        ],
    )
    def k(table_hbm, idx_hbm, out_hbm, idx_v, rows_v, sem):
        wid = lax.axis_index("s") * info.num_cores + lax.axis_index("c")
        base = wid * b_per_w
        pltpu.sync_copy(idx_hbm.at[wid], idx_v)
        copies = [
            pltpu.async_copy(table_hbm.at[idx_v.at[j]],
                             rows_v.at[pl.ds(j * 128, 128)], sem)
            for j in range(kch)
        ]
        for c in copies:
            c.wait()
        pltpu.sync_copy(rows_v, out_hbm.at[pl.ds(base, b_per_w)])

    return k(table, idx3)[:, :d_orig]


def kernel(x, embedding_weight):
    e_dim = x.shape[-1]
    latent = x.reshape(-1, e_dim)
    sqx = jnp.sum(latent ** 2, axis=1, keepdims=True)
    sqe = jnp.sum(embedding_weight ** 2, axis=1, keepdims=True).T
    idx, dsum = _distance_argmin(latent, sqx, -2.0 * embedding_weight, sqe)
    x_q = _sc_gather(embedding_weight, idx)
    loss = (1.0 + _BETA) * dsum / jnp.float32(latent.size)
    return (x_q.reshape(x.shape), loss, idx.reshape(x.shape[:-1]))


# speedup vs baseline: 1.2483x; 1.2483x over previous
import functools

import jax
import jax.numpy as jnp
from jax import lax
from jax.experimental import pallas as pl
from jax.experimental.pallas import tpu as pltpu
from jax.experimental.pallas import tpu_sc as plsc

_BETA = 0.25
_T = 2048
_KT = 1024


def _combine(lhs, rhs, rounded):
    lv, li = lhs
    rv, ri = rhs
    if rounded:
        u = lax.bitcast_convert_type(lv, jnp.uint32)
        u = (u + jnp.uint32(0x7FFF) + ((u >> 16) & jnp.uint32(1))) \
            & jnp.uint32(0xFFFF0000)
        cmp_l = lax.bitcast_convert_type(u, jnp.float32)
    else:
        cmp_l = lv
    keep = cmp_l <= rv
    return jnp.where(keep, lv, rv), jnp.where(keep, li, ri)


def _argmin_body(n_e, x_ref, sqx_ref, e_ref, sqe_ref, idx_ref, dsum_ref):
    x = x_ref[...]
    xb = x.astype(jnp.bfloat16)
    sqx = sqx_ref[...]
    champs = []
    for c in range(n_e // _KT):
        e_c = e_ref[c * _KT:(c + 1) * _KT, :].astype(jnp.bfloat16)
        sqe_c = sqe_ref[:, c * _KT:(c + 1) * _KT]
        s2 = lax.dot_general(xb, e_c, (((1,), (1,)), ((), ())),
                             preferred_element_type=jnp.float32)
        d = (sqx + sqe_c) + s2
        cm = jnp.min(d, axis=1, keepdims=True)
        io = lax.broadcasted_iota(jnp.int32, (_T, _KT), 1)
        ci = jnp.min(jnp.where(d == cm, io, jnp.int32(2**31 - 1)),
                     axis=1, keepdims=True) + c * _KT
        champs.append((cm, ci))
    a = _combine(champs[0], champs[1], False)
    c = _combine(champs[2], champs[3], False)
    dd = _combine(champs[4], champs[5], False)
    b = _combine(champs[6], champs[7], False)
    a2 = _combine(a, c, False)
    b2 = _combine(dd, b, False)
    m, bi = _combine(a2, b2, True)
    idx_ref[...] = bi

    @pl.when(pl.program_id(0) == 0)
    def _():
        dsum_ref[...] = jnp.zeros_like(dsum_ref)

    dsum_ref[...] += jnp.sum(m, axis=0, keepdims=True)


def _distance_argmin(latent, sqx, emb, sqe):
    n, e_dim = latent.shape
    n_e = emb.shape[0]
    grid = n // _T
    idx, dsum = pl.pallas_call(
        functools.partial(_argmin_body, n_e),
        grid=(grid,),
        in_specs=[
            pl.BlockSpec((_T, e_dim), lambda i: (i, 0)),
            pl.BlockSpec((_T, 1), lambda i: (i, 0)),
            pl.BlockSpec((n_e, e_dim), lambda i: (0, 0)),
            pl.BlockSpec((1, n_e), lambda i: (0, 0)),
        ],
        out_specs=[
            pl.BlockSpec((_T, 1), lambda i: (i, 0)),
            pl.BlockSpec((1, 1), lambda i: (0, 0)),
        ],
        out_shape=[
            jax.ShapeDtypeStruct((n, 1), jnp.int32),
            jax.ShapeDtypeStruct((1, 1), jnp.float32),
        ],
    )(latent, sqx, emb, sqe)
    return idx.reshape(n), dsum[0, 0]


def _sc_gather(table, idx):
    b = idx.shape[0]
    d_orig = table.shape[1]
    table = jnp.pad(table, ((0, 0), (0, 128 - d_orig)))
    d = 128
    info = plsc.get_sparse_core_info()
    nw = info.num_cores * info.num_subcores
    b_per_w = b // nw
    kch = b_per_w // 128
    idx3 = idx.reshape(nw, kch, 128)
    mesh = plsc.VectorSubcoreMesh(core_axis_name="c", subcore_axis_name="s")

    @functools.partial(
        pl.kernel,
        mesh=mesh,
        out_type=jax.ShapeDtypeStruct((b, d), jnp.float32),
        scratch_types=[
            pltpu.VMEM((kch, 128), jnp.int32),
            pltpu.VMEM((b_per_w, d), jnp.float32),
            pltpu.SemaphoreType.DMA,
        ],
    )
    def k(table_hbm, idx_hbm, out_hbm, idx_v, rows_v, sem):
        wid = lax.axis_index("s") * info.num_cores + lax.axis_index("c")
        base = wid * b_per_w
        pltpu.sync_copy(idx_hbm.at[wid], idx_v)
        copies = [
            pltpu.async_copy(table_hbm.at[idx_v.at[j]],
                             rows_v.at[pl.ds(j * 128, 128)], sem)
            for j in range(kch)
        ]
        for c in copies:
            c.wait()
        pltpu.sync_copy(rows_v, out_hbm.at[pl.ds(base, b_per_w)])

    return k(table, idx3)[:, :d_orig]


def kernel(x, embedding_weight):
    e_dim = x.shape[-1]
    latent = x.reshape(-1, e_dim)
    sqx = jnp.sum(latent ** 2, axis=1, keepdims=True)
    sqe = jnp.sum(embedding_weight ** 2, axis=1, keepdims=True).T
    idx, dsum = _distance_argmin(latent, sqx, -2.0 * embedding_weight, sqe)
    x_q = _sc_gather(embedding_weight, idx)
    loss = (1.0 + _BETA) * dsum / jnp.float32(latent.size)
    return (x_q.reshape(x.shape), loss, idx.reshape(x.shape[:-1]))
